# baseline (speedup 1.0000x reference)
import functools

import jax
import jax.numpy as jnp
from jax import lax
from jax.experimental import pallas as pl
from jax.experimental.pallas import tpu as pltpu, tpu_sc as plsc

N = 100000
E = 3200000
D_IN = 128
EPS = 0.5

NP = 100096
NTILES = 16
ECORE = E // (2 * NTILES)
ETILE = E // NTILES
EBLK = 2000

_mesh = lambda: plsc.VectorSubcoreMesh(core_axis_name="c", subcore_axis_name="s")


def _deg_body(col_hbm, out_hbm, idx_v, idxt_v, ones_v, zstage_v, acc_sh, sem):
    core = lax.axis_index("c")
    sub = lax.axis_index("s")
    zeros16 = jnp.zeros((16,), jnp.float32)
    ones16 = jnp.ones((16,), jnp.float32)
    for k in range(8):
        ones_v[pl.ds(16 * k, 16)] = ones16
    for k in range(64):
        zstage_v[pl.ds(16 * k, 16)] = zeros16
    zbase = sub * (NP // NTILES)
    for off in (0, 1024, 2048, 3072, 4096, 5120):
        pltpu.sync_copy(zstage_v, acc_sh.at[pl.ds(zbase + off, 1024)])
    pltpu.sync_copy(zstage_v.at[pl.ds(0, 112)], acc_sh.at[pl.ds(zbase + 6144, 112)])
    plsc.subcore_barrier()

    base_e = (core * NTILES + sub) * ECORE

    def body(b, _):
        pltpu.sync_copy(col_hbm.at[pl.ds(base_e + b * 128, 128)], idx_v)
        pltpu.sync_copy(ones_v, acc_sh.at[idx_v], add=True)
        return 0

    lax.fori_loop(0, ECORE // 128, body, 0)
    pltpu.sync_copy(col_hbm.at[pl.ds(base_e + 781 * 128, 32)], idxt_v)
    pltpu.sync_copy(ones_v.at[pl.ds(0, 32)], acc_sh.at[idxt_v], add=True)
    plsc.subcore_barrier()

    @pl.when(sub == 0)
    def _():
        pltpu.sync_copy(acc_sh, out_hbm.at[core])


_deg_kernel = functools.partial(
    pl.kernel,
    mesh=_mesh(),
    out_type=jax.ShapeDtypeStruct((2, NP), jnp.float32),
    scratch_types=[
        pltpu.VMEM((128,), jnp.int32),
        pltpu.VMEM((32,), jnp.int32),
        pltpu.VMEM((128,), jnp.float32),
        pltpu.VMEM((1024,), jnp.float32),
        pltpu.VMEM_SHARED((NP,), jnp.float32),
        pltpu.SemaphoreType.DMA,
    ],
)(_deg_body)


PC = 7168
PASSES = 7
PCAP = 15872
SROWS = PASSES * 2 * PC
TPT = PC // NTILES


def _make_propagate(HALVES, mode):
    def body(*refs):
        i = HALVES
        u_hbms = refs[:HALVES]
        if mode == "save":
            row_hbm, col_hbm = refs[i], refs[i + 1]
            i += 2
        else:
            cl_hbm, rl_hbm, cn_hbm = refs[i], refs[i + 1], refs[i + 2]
            i += 3
        s_hbms = refs[i:i + HALVES]
        i += HALVES
        if mode == "save":
            cl_hbm, rl_hbm, cn_hbm = refs[i], refs[i + 1], refs[i + 2]
            i += 3
        (cstage, rstage, cbuf, rbuf, staging, zbuf, idxg, idxs0, idxs1,
         cntbuf, acc, gsem0, gsem1, ssem0, ssem1) = refs[i:]
        core = lax.axis_index("c")
        sub = lax.axis_index("s")
        zeros16 = jnp.zeros((16,), jnp.float32)
        for r in range(8):
            for f in range(8):
                zbuf[r, pl.ds(16 * f, 16)] = zeros16
        zoff = sub * TPT

        def zero_slice(j, _):
            pltpu.sync_copy(zbuf, acc.at[pl.ds(zoff + 8 * j, 8)])
            return 0

        lax.fori_loop(0, TPT // 8, zero_slice, 0)
        if mode == "reuse":
            pltpu.sync_copy(cn_hbm.at[core].at[sub], cntbuf)
        plsc.subcore_barrier()

        def one_pass(p, _):
            base = (p * 2 + core) * PC

            if mode == "save":
                ebase = sub * ETILE

                def blk_body(b, cntv):
                    off = ebase + b * EBLK
                    pltpu.sync_copy(col_hbm.at[pl.ds(off, EBLK)], cstage)
                    pltpu.sync_copy(row_hbm.at[pl.ds(off, EBLK)], rstage)

                    def f_body(i, cntv):
                        c16 = cstage[pl.ds(i * 16, 16)]
                        r16 = rstage[pl.ds(i * 16, 16)]
                        lo = c16 - base
                        m = (lo >= 0) & (lo < PC)
                        m32 = m.astype(jnp.int32)
                        csum = plsc.cumsum(m32)
                        pos = csum - m32 + cntv
                        plsc.store_scatter(cbuf, [pos], lo, mask=m)
                        plsc.store_scatter(rbuf, [pos], r16, mask=m)
                        pc = plsc.all_reduce_population_count(m)
                        return jnp.minimum(cntv + pc, PCAP - 144)

                    return lax.fori_loop(0, EBLK // 16, f_body, cntv)

                cntv = lax.fori_loop(0, ETILE // EBLK, blk_body,
                                     jnp.zeros((16,), jnp.int32))
                lane0 = lax.iota(jnp.int32, 16)
                cnt = jnp.sum(jnp.where(lane0 == 0, cntv, 0))

                lane = lax.iota(jnp.int32, 16)
                wid = core * NTILES + sub
                padr = wid * 3000 + lane * 64
                padc = PC + lane
                for k in range(8):
                    cbuf[pl.ds(cnt + 16 * k, 16)] = padc
                    rbuf[pl.ds(cnt + 16 * k, 16)] = padr
                plsc.store_scatter(cntbuf, [jnp.full((16,), p, jnp.int32)],
                                   jnp.full((16,), cnt, jnp.int32),
                                   mask=lane == p)
                pltpu.sync_copy(cbuf, cl_hbm.at[core].at[sub].at[p])
                pltpu.sync_copy(rbuf, rl_hbm.at[core].at[sub].at[p])
            else:
                pltpu.sync_copy(cl_hbm.at[core].at[sub].at[p], cbuf)
                pltpu.sync_copy(rl_hbm.at[core].at[sub].at[p], rbuf)
                cv = cntbuf[pl.ds(0, 16)]
                cnt = jnp.sum(jnp.where(lax.iota(jnp.int32, 16) == p, cv, 0))
            nb = (cnt + 127) // 128

            for h in range(HALVES):
                u_hbm = u_hbms[h]
                gsems = (gsem0, gsem1)
                ssems = (ssem0, ssem1)
                sidx = (idxs0, idxs1)

                def load_gidx(b, slot):
                    for k in range(8):
                        idxg[pl.ds(slot * 128 + 16 * k, 16)] = (
                            rbuf[pl.ds(b * 128 + 16 * k, 16)])

                def fire_g(slot):
                    pltpu.async_copy(
                        u_hbm.at[idxg.at[pl.ds(slot * 128, 128)]],
                        staging.at[pl.ds(slot * 128, 128)], gsems[slot])

                def wait_g(slot):
                    pltpu.make_async_copy(
                        u_hbm.at[idxg.at[pl.ds(slot * 128, 128)]],
                        staging.at[pl.ds(slot * 128, 128)], gsems[slot]).wait()

                def fire_s(slot):
                    pltpu.async_copy(staging.at[pl.ds(slot * 128, 128)],
                                     acc.at[sidx[slot]], ssems[slot], add=True)

                def wait_s(slot):
                    pltpu.make_async_copy(staging.at[pl.ds(slot * 128, 128)],
                                          acc.at[sidx[slot]],
                                          ssems[slot]).wait()

                @pl.when(nb > 0)
                def _():
                    load_gidx(0, 0)
                    fire_g(0)

                def batch(b, _):
                    s = b % 2

                    @pl.when(b >= 1)
                    def _():
                        @pl.when(s == 0)
                        def _():
                            wait_s(1)

                        @pl.when(s == 1)
                        def _():
                            wait_s(0)

                    @pl.when(b + 1 < nb)
                    def _():
                        @pl.when(s == 0)
                        def _():
                            load_gidx(b + 1, 1)
                            fire_g(1)

                        @pl.when(s == 1)
                        def _():
                            load_gidx(b + 1, 0)
                            fire_g(0)

                    @pl.when(s == 0)
                    def _():
                        wait_g(0)
                        for k in range(8):
                            idxs0[pl.ds(16 * k, 16)] = (
                                cbuf[pl.ds(b * 128 + 16 * k, 16)])
                        fire_s(0)

                    @pl.when(s == 1)
                    def _():
                        wait_g(1)
                        for k in range(8):
                            idxs1[pl.ds(16 * k, 16)] = (
                                cbuf[pl.ds(b * 128 + 16 * k, 16)])
                        fire_s(1)
                    return 0

                lax.fori_loop(0, nb, batch, 0)

                @pl.when(nb > 0)
                def _():
                    @pl.when((nb - 1) % 2 == 0)
                    def _():
                        wait_s(0)

                    @pl.when((nb - 1) % 2 == 1)
                    def _():
                        wait_s(1)

                plsc.subcore_barrier()
                pltpu.sync_copy(acc.at[pl.ds(zoff, TPT)],
                                s_hbms[h].at[pl.ds(base + zoff, TPT)])
                lax.fori_loop(0, TPT // 8, zero_slice, 0)
                plsc.subcore_barrier()
            return 0

        lax.fori_loop(0, PASSES, one_pass, 0)
        if mode == "save":
            pltpu.sync_copy(cntbuf, cn_hbm.at[core].at[sub])

    s_t = [jax.ShapeDtypeStruct((SROWS, 128), jnp.float32)] * HALVES
    lists_t = [
        jax.ShapeDtypeStruct((2, NTILES, 8, PCAP), jnp.int32),
        jax.ShapeDtypeStruct((2, NTILES, 8, PCAP), jnp.int32),
        jax.ShapeDtypeStruct((2, NTILES, 16), jnp.int32),
    ]
    out_t = s_t + lists_t if mode == "save" else (
        s_t if HALVES > 1 else s_t[0])
    return functools.partial(
        pl.kernel,
        mesh=_mesh(),
        out_type=out_t,
        scratch_types=[
            pltpu.VMEM((EBLK,), jnp.int32),
            pltpu.VMEM((EBLK,), jnp.int32),
            pltpu.VMEM((PCAP,), jnp.int32),
            pltpu.VMEM((PCAP,), jnp.int32),
            pltpu.VMEM((256, 128), jnp.float32),
            pltpu.VMEM((8, 128), jnp.float32),
            pltpu.VMEM((256,), jnp.int32),
            pltpu.VMEM((128,), jnp.int32),
            pltpu.VMEM((128,), jnp.int32),
            pltpu.VMEM((16,), jnp.int32),
            pltpu.VMEM_SHARED((PC + 16, 128), jnp.float32),
            pltpu.SemaphoreType.DMA,
            pltpu.SemaphoreType.DMA,
            pltpu.SemaphoreType.DMA,
            pltpu.SemaphoreType.DMA,
        ],
        compiler_params=pltpu.CompilerParams(needs_layout_passes=False),
    )(body)


_prop2_save = _make_propagate(2, "save")
_prop2_reuse = _make_propagate(2, "reuse")
_prop1_save = _make_propagate(1, "save")
_prop1_reuse = _make_propagate(1, "reuse")


_RB = 200
_GRID = N // _RB
_bs = lambda w: pl.BlockSpec((_RB, w), lambda i: (i, 0))


def _mm_body(x_ref, w_ref, b_ref, disp_ref, diss_ref,
             h0c_ref, h0s_ref, u0a_ref, u0b_ref, u0s_ref):
    h = jnp.dot(x_ref[...], w_ref[...], preferred_element_type=jnp.float32)
    h = h + b_ref[...]
    hb = h[:, 128:]
    h0c_ref[...] = h
    h0s_ref[...] = hb
    u0a_ref[...] = disp_ref[...] * h[:, :128]
    u0b_ref[...] = disp_ref[...] * hb
    u0s_ref[...] = diss_ref[...] * hb


def _mm_call(x, wc, bc, disp, diss):
    return pl.pallas_call(
        _mm_body,
        grid=(_GRID,),
        in_specs=[
            _bs(128),
            pl.BlockSpec((128, 256), lambda i: (0, 0)),
            pl.BlockSpec((1, 256), lambda i: (0, 0)),
            _bs(1),
            _bs(1),
        ],
        out_specs=[_bs(256), _bs(128), _bs(128), _bs(128), _bs(128)],
        out_shape=[
            jax.ShapeDtypeStruct((N, 256), jnp.float32),
            jax.ShapeDtypeStruct((N, 128), jnp.float32),
            jax.ShapeDtypeStruct((N, 128), jnp.float32),
            jax.ShapeDtypeStruct((N, 128), jnp.float32),
            jax.ShapeDtypeStruct((N, 128), jnp.float32),
        ],
    )(x, wc, bc, disp, diss)


def _norm_body(pp_ref, sp_ref, disp_ref, ap_ref, diss_ref, as_ref):
    for pref, dref, aref in ((pp_ref, disp_ref, ap_ref),
                             (sp_ref, diss_ref, as_ref)):
        deg = pref[:782, :] + pref[782:, :] + EPS
        dis = lax.rsqrt(deg)
        dref[...] = dis
        aref[...] = (1.0 - EPS) + EPS * dis * dis


def _norm_call(pparts, sparts):
    return pl.pallas_call(
        _norm_body,
        grid=(1,),
        in_specs=[pl.BlockSpec((1564, 128), lambda i: (0, 0))] * 2,
        out_specs=[pl.BlockSpec((782, 128), lambda i: (0, 0))] * 4,
        out_shape=[jax.ShapeDtypeStruct((782, 128), jnp.float32)] * 4,
    )(pparts, sparts)


def _combine_c_body(h_ref, sa_ref, sb_ref, a_ref, dis_ref,
                    h1_ref, u1a_ref, u1b_ref):
    s = jnp.concatenate([sa_ref[...], sb_ref[...]], axis=1)
    h1 = a_ref[...] * h_ref[...] + dis_ref[...] * s
    h1_ref[...] = h1
    u1a_ref[...] = dis_ref[...] * h1[:, :128]
    u1b_ref[...] = dis_ref[...] * h1[:, 128:]


def _combine_c(h, sa, sb, a2, dis2):
    return pl.pallas_call(
        _combine_c_body, grid=(_GRID,),
        in_specs=[_bs(256), _bs(128), _bs(128), _bs(1), _bs(1)],
        out_specs=[_bs(256), _bs(128), _bs(128)],
        out_shape=[jax.ShapeDtypeStruct((N, 256), jnp.float32)]
        + [jax.ShapeDtypeStruct((N, 128), jnp.float32)] * 2,
    )(h, sa, sb, a2, dis2)


def _final_c_body(h_ref, sa_ref, sb_ref, a_ref, dis_ref, z1_ref, t2_ref):
    z1_ref[...] = a_ref[...] * h_ref[:, :128] + dis_ref[...] * sa_ref[...]
    t2_ref[...] = a_ref[...] * h_ref[:, 128:] + dis_ref[...] * sb_ref[...]


def _final_c(h, sa, sb, a2, dis2):
    return pl.pallas_call(
        _final_c_body, grid=(_GRID,),
        in_specs=[_bs(256), _bs(128), _bs(128), _bs(1), _bs(1)],
        out_specs=[_bs(128), _bs(128)],
        out_shape=[jax.ShapeDtypeStruct((N, 128), jnp.float32)] * 2,
    )(h, sa, sb, a2, dis2)


def _combine_s_body(h_ref, s_ref, a_ref, dis_ref, h1_ref, u1_ref):
    h1 = a_ref[...] * h_ref[...] + dis_ref[...] * s_ref[...]
    h1_ref[...] = h1
    u1_ref[...] = dis_ref[...] * h1


def _combine_s(h, s, a2, dis2):
    return pl.pallas_call(
        _combine_s_body, grid=(_GRID,),
        in_specs=[_bs(128), _bs(128), _bs(1), _bs(1)],
        out_specs=[_bs(128), _bs(128)],
        out_shape=[jax.ShapeDtypeStruct((N, 128), jnp.float32)] * 2,
    )(h, s, a2, dis2)


def _final_s_body(h_ref, s_ref, a_ref, dis_ref, t2_ref, z2_ref):
    z2_ref[...] = (a_ref[...] * h_ref[...] + dis_ref[...] * s_ref[...]
                   + t2_ref[...])


def _final_s(h, s, a2, dis2, t2):
    return pl.pallas_call(
        _final_s_body, grid=(_GRID,),
        in_specs=[_bs(128), _bs(128), _bs(1), _bs(1), _bs(128)],
        out_specs=_bs(128),
        out_shape=jax.ShapeDtypeStruct((N, 128), jnp.float32),
    )(h, s, a2, dis2, t2)


def kernel(x, pri_edges, sup_edges, W1, b1, W2, b2):
    pe = pri_edges.astype(jnp.int32)
    se = sup_edges.astype(jnp.int32)
    prow, pcol = pe[0], pe[1]
    srow, scol = se[0], se[1]

    degp = _deg_kernel(pcol).reshape(1564, 128)
    degs = _deg_kernel(scol).reshape(1564, 128)
    disp, ap, diss, as_ = _norm_call(degp, degs)

    def col2d(v):
        return v.reshape(NP)[:N].reshape(N, 1)

    disp2, ap2, diss2, as2 = map(col2d, (disp, ap, diss, as_))

    wc = jnp.concatenate([W1, W2], axis=1)
    bc = jnp.concatenate([b1, b2]).reshape(1, 256)
    h0c, h0s, u0a, u0b, u0s = _mm_call(x, wc, bc, disp2, diss2)

    s1a, s1b, pcl, prl, pcn = _prop2_save(u0a, u0b, prow, pcol)
    h1c, u1a, u1b = _combine_c(h0c, s1a, s1b, ap2, disp2)
    s2a, s2b = _prop2_reuse(u1a, u1b, pcl, prl, pcn)
    z1, t2 = _final_c(h1c, s2a, s2b, ap2, disp2)

    s1s, scl, srl, scn = _prop1_save(u0s, srow, scol)
    h1s, u1s = _combine_s(h0s, s1s, as2, diss2)
    s2s = _prop1_reuse(u1s, scl, srl, scn)
    z2 = _final_s(h1s, s2s, as2, diss2, t2)
    return (z1, z2)


# speedup vs baseline: 1.0236x; 1.0236x over previous
import functools

import jax
import jax.numpy as jnp
from jax import lax
from jax.experimental import pallas as pl
from jax.experimental.pallas import tpu as pltpu, tpu_sc as plsc

N = 100000
E = 3200000
D_IN = 128
EPS = 0.5

NP = 100096
NTILES = 16
ECORE = E // (2 * NTILES)
ETILE = E // NTILES
EBLK = 2000

_mesh = lambda: plsc.VectorSubcoreMesh(core_axis_name="c", subcore_axis_name="s")


def _deg_body(col_hbm, out_hbm, idx_v, idxt_v, ones_v, zstage_v, acc_sh, sem):
    core = lax.axis_index("c")
    sub = lax.axis_index("s")
    zeros16 = jnp.zeros((16,), jnp.float32)
    ones16 = jnp.ones((16,), jnp.float32)
    for k in range(8):
        ones_v[pl.ds(16 * k, 16)] = ones16
    for k in range(64):
        zstage_v[pl.ds(16 * k, 16)] = zeros16
    zbase = sub * (NP // NTILES)
    for off in (0, 1024, 2048, 3072, 4096, 5120):
        pltpu.sync_copy(zstage_v, acc_sh.at[pl.ds(zbase + off, 1024)])
    pltpu.sync_copy(zstage_v.at[pl.ds(0, 112)], acc_sh.at[pl.ds(zbase + 6144, 112)])
    plsc.subcore_barrier()

    base_e = (core * NTILES + sub) * ECORE

    def body(b, _):
        pltpu.sync_copy(col_hbm.at[pl.ds(base_e + b * 128, 128)], idx_v)
        pltpu.sync_copy(ones_v, acc_sh.at[idx_v], add=True)
        return 0

    lax.fori_loop(0, ECORE // 128, body, 0)
    pltpu.sync_copy(col_hbm.at[pl.ds(base_e + 781 * 128, 32)], idxt_v)
    pltpu.sync_copy(ones_v.at[pl.ds(0, 32)], acc_sh.at[idxt_v], add=True)
    plsc.subcore_barrier()

    @pl.when(sub == 0)
    def _():
        pltpu.sync_copy(acc_sh, out_hbm.at[core])


_deg_kernel = functools.partial(
    pl.kernel,
    mesh=_mesh(),
    out_type=jax.ShapeDtypeStruct((2, NP), jnp.float32),
    scratch_types=[
        pltpu.VMEM((128,), jnp.int32),
        pltpu.VMEM((32,), jnp.int32),
        pltpu.VMEM((128,), jnp.float32),
        pltpu.VMEM((1024,), jnp.float32),
        pltpu.VMEM_SHARED((NP,), jnp.float32),
        pltpu.SemaphoreType.DMA,
    ],
)(_deg_body)


PC = 7168
PASSES = 7
PCAP = 15872
SROWS = PASSES * 2 * PC
TPT = PC // NTILES


def _make_propagate(HALVES, mode):
    def body(*refs):
        i = HALVES
        u_hbms = refs[:HALVES]
        if mode == "save":
            row_hbm, col_hbm = refs[i], refs[i + 1]
            i += 2
        else:
            cl_hbm, rl_hbm, cn_hbm = refs[i], refs[i + 1], refs[i + 2]
            i += 3
        s_hbms = refs[i:i + HALVES]
        i += HALVES
        if mode == "save":
            cl_hbm, rl_hbm, cn_hbm = refs[i], refs[i + 1], refs[i + 2]
            i += 3
        (cstage, rstage, cbuf, rbuf, staging, zbuf, idxg, idxs0, idxs1,
         cntbuf, acc, gsem0, gsem1, ssem0, ssem1) = refs[i:]
        core = lax.axis_index("c")
        sub = lax.axis_index("s")
        zeros16 = jnp.zeros((16,), jnp.float32)
        for r in range(8):
            for f in range(8):
                zbuf[r, pl.ds(16 * f, 16)] = zeros16
        zoff = sub * TPT

        def zero_slice(j, _):
            pltpu.sync_copy(zbuf, acc.at[pl.ds(zoff + 8 * j, 8)])
            return 0

        lax.fori_loop(0, TPT // 8, zero_slice, 0)
        if mode == "reuse":
            pltpu.sync_copy(cn_hbm.at[core].at[sub], cntbuf)
        plsc.subcore_barrier()

        def one_pass(p, _):
            base = (p * 2 + core) * PC

            if mode == "save":
                ebase = sub * ETILE

                def blk_body(b, cntv):
                    off = ebase + b * EBLK
                    pltpu.sync_copy(col_hbm.at[pl.ds(off, EBLK)], cstage)
                    pltpu.sync_copy(row_hbm.at[pl.ds(off, EBLK)], rstage)

                    def f_body(i, cntv):
                        for u in range(5):
                            c16 = cstage[pl.ds(i * 80 + u * 16, 16)]
                            r16 = rstage[pl.ds(i * 80 + u * 16, 16)]
                            lo = c16 - base
                            m = (lo >= 0) & (lo < PC)
                            m32 = m.astype(jnp.int32)
                            csum = plsc.cumsum(m32)
                            pos = csum - m32 + cntv
                            plsc.store_scatter(cbuf, [pos], lo, mask=m)
                            plsc.store_scatter(rbuf, [pos], r16, mask=m)
                            cntv = cntv + plsc.all_reduce_population_count(m)
                        return jnp.minimum(cntv, PCAP - 224)

                    return lax.fori_loop(0, EBLK // 80, f_body, cntv)

                cntv = lax.fori_loop(0, ETILE // EBLK, blk_body,
                                     jnp.zeros((16,), jnp.int32))
                lane0 = lax.iota(jnp.int32, 16)
                cnt = jnp.sum(jnp.where(lane0 == 0, cntv, 0))

                lane = lax.iota(jnp.int32, 16)
                wid = core * NTILES + sub
                padr = wid * 3000 + lane * 64
                padc = PC + lane
                for k in range(8):
                    cbuf[pl.ds(cnt + 16 * k, 16)] = padc
                    rbuf[pl.ds(cnt + 16 * k, 16)] = padr
                plsc.store_scatter(cntbuf, [jnp.full((16,), p, jnp.int32)],
                                   jnp.full((16,), cnt, jnp.int32),
                                   mask=lane == p)
                pltpu.sync_copy(cbuf, cl_hbm.at[core].at[sub].at[p])
                pltpu.sync_copy(rbuf, rl_hbm.at[core].at[sub].at[p])
            else:
                pltpu.sync_copy(cl_hbm.at[core].at[sub].at[p], cbuf)
                pltpu.sync_copy(rl_hbm.at[core].at[sub].at[p], rbuf)
                cv = cntbuf[pl.ds(0, 16)]
                cnt = jnp.sum(jnp.where(lax.iota(jnp.int32, 16) == p, cv, 0))
            nb = (cnt + 127) // 128

            for h in range(HALVES):
                u_hbm = u_hbms[h]
                gsems = (gsem0, gsem1)
                ssems = (ssem0, ssem1)
                sidx = (idxs0, idxs1)

                def load_gidx(b, slot):
                    for k in range(8):
                        idxg[pl.ds(slot * 128 + 16 * k, 16)] = (
                            rbuf[pl.ds(b * 128 + 16 * k, 16)])

                def fire_g(slot):
                    pltpu.async_copy(
                        u_hbm.at[idxg.at[pl.ds(slot * 128, 128)]],
                        staging.at[pl.ds(slot * 128, 128)], gsems[slot])

                def wait_g(slot):
                    pltpu.make_async_copy(
                        u_hbm.at[idxg.at[pl.ds(slot * 128, 128)]],
                        staging.at[pl.ds(slot * 128, 128)], gsems[slot]).wait()

                def fire_s(slot):
                    pltpu.async_copy(staging.at[pl.ds(slot * 128, 128)],
                                     acc.at[sidx[slot]], ssems[slot], add=True)

                def wait_s(slot):
                    pltpu.make_async_copy(staging.at[pl.ds(slot * 128, 128)],
                                          acc.at[sidx[slot]],
                                          ssems[slot]).wait()

                @pl.when(nb > 0)
                def _():
                    load_gidx(0, 0)
                    fire_g(0)

                def batch(b, _):
                    s = b % 2

                    @pl.when(b >= 1)
                    def _():
                        @pl.when(s == 0)
                        def _():
                            wait_s(1)

                        @pl.when(s == 1)
                        def _():
                            wait_s(0)

                    @pl.when(b + 1 < nb)
                    def _():
                        @pl.when(s == 0)
                        def _():
                            load_gidx(b + 1, 1)
                            fire_g(1)

                        @pl.when(s == 1)
                        def _():
                            load_gidx(b + 1, 0)
                            fire_g(0)

                    @pl.when(s == 0)
                    def _():
                        wait_g(0)
                        for k in range(8):
                            idxs0[pl.ds(16 * k, 16)] = (
                                cbuf[pl.ds(b * 128 + 16 * k, 16)])
                        fire_s(0)

                    @pl.when(s == 1)
                    def _():
                        wait_g(1)
                        for k in range(8):
                            idxs1[pl.ds(16 * k, 16)] = (
                                cbuf[pl.ds(b * 128 + 16 * k, 16)])
                        fire_s(1)
                    return 0

                lax.fori_loop(0, nb, batch, 0)

                @pl.when(nb > 0)
                def _():
                    @pl.when((nb - 1) % 2 == 0)
                    def _():
                        wait_s(0)

                    @pl.when((nb - 1) % 2 == 1)
                    def _():
                        wait_s(1)

                plsc.subcore_barrier()
                pltpu.sync_copy(acc.at[pl.ds(zoff, TPT)],
                                s_hbms[h].at[pl.ds(base + zoff, TPT)])
                lax.fori_loop(0, TPT // 8, zero_slice, 0)
                plsc.subcore_barrier()
            return 0

        lax.fori_loop(0, PASSES, one_pass, 0)
        if mode == "save":
            pltpu.sync_copy(cntbuf, cn_hbm.at[core].at[sub])

    s_t = [jax.ShapeDtypeStruct((SROWS, 128), jnp.float32)] * HALVES
    lists_t = [
        jax.ShapeDtypeStruct((2, NTILES, 8, PCAP), jnp.int32),
        jax.ShapeDtypeStruct((2, NTILES, 8, PCAP), jnp.int32),
        jax.ShapeDtypeStruct((2, NTILES, 16), jnp.int32),
    ]
    out_t = s_t + lists_t if mode == "save" else (
        s_t if HALVES > 1 else s_t[0])
    return functools.partial(
        pl.kernel,
        mesh=_mesh(),
        out_type=out_t,
        scratch_types=[
            pltpu.VMEM((EBLK,), jnp.int32),
            pltpu.VMEM((EBLK,), jnp.int32),
            pltpu.VMEM((PCAP,), jnp.int32),
            pltpu.VMEM((PCAP,), jnp.int32),
            pltpu.VMEM((256, 128), jnp.float32),
            pltpu.VMEM((8, 128), jnp.float32),
            pltpu.VMEM((256,), jnp.int32),
            pltpu.VMEM((128,), jnp.int32),
            pltpu.VMEM((128,), jnp.int32),
            pltpu.VMEM((16,), jnp.int32),
            pltpu.VMEM_SHARED((PC + 16, 128), jnp.float32),
            pltpu.SemaphoreType.DMA,
            pltpu.SemaphoreType.DMA,
            pltpu.SemaphoreType.DMA,
            pltpu.SemaphoreType.DMA,
        ],
        compiler_params=pltpu.CompilerParams(needs_layout_passes=False),
    )(body)


_prop2_save = _make_propagate(2, "save")
_prop2_reuse = _make_propagate(2, "reuse")
_prop1_save = _make_propagate(1, "save")
_prop1_reuse = _make_propagate(1, "reuse")


_RB = 200
_GRID = N // _RB
_bs = lambda w: pl.BlockSpec((_RB, w), lambda i: (i, 0))


def _mm_body(x_ref, w_ref, b_ref, disp_ref, diss_ref,
             h0c_ref, h0s_ref, u0a_ref, u0b_ref, u0s_ref):
    h = jnp.dot(x_ref[...], w_ref[...], preferred_element_type=jnp.float32)
    h = h + b_ref[...]
    hb = h[:, 128:]
    h0c_ref[...] = h
    h0s_ref[...] = hb
    u0a_ref[...] = disp_ref[...] * h[:, :128]
    u0b_ref[...] = disp_ref[...] * hb
    u0s_ref[...] = diss_ref[...] * hb


def _mm_call(x, wc, bc, disp, diss):
    return pl.pallas_call(
        _mm_body,
        grid=(_GRID,),
        in_specs=[
            _bs(128),
            pl.BlockSpec((128, 256), lambda i: (0, 0)),
            pl.BlockSpec((1, 256), lambda i: (0, 0)),
            _bs(1),
            _bs(1),
        ],
        out_specs=[_bs(256), _bs(128), _bs(128), _bs(128), _bs(128)],
        out_shape=[
            jax.ShapeDtypeStruct((N, 256), jnp.float32),
            jax.ShapeDtypeStruct((N, 128), jnp.float32),
            jax.ShapeDtypeStruct((N, 128), jnp.float32),
            jax.ShapeDtypeStruct((N, 128), jnp.float32),
            jax.ShapeDtypeStruct((N, 128), jnp.float32),
        ],
    )(x, wc, bc, disp, diss)


def _norm_body(pp_ref, sp_ref, disp_ref, ap_ref, diss_ref, as_ref):
    for pref, dref, aref in ((pp_ref, disp_ref, ap_ref),
                             (sp_ref, diss_ref, as_ref)):
        deg = pref[:782, :] + pref[782:, :] + EPS
        dis = lax.rsqrt(deg)
        dref[...] = dis
        aref[...] = (1.0 - EPS) + EPS * dis * dis


def _norm_call(pparts, sparts):
    return pl.pallas_call(
        _norm_body,
        grid=(1,),
        in_specs=[pl.BlockSpec((1564, 128), lambda i: (0, 0))] * 2,
        out_specs=[pl.BlockSpec((782, 128), lambda i: (0, 0))] * 4,
        out_shape=[jax.ShapeDtypeStruct((782, 128), jnp.float32)] * 4,
    )(pparts, sparts)


def _combine_c_body(h_ref, sa_ref, sb_ref, a_ref, dis_ref,
                    h1_ref, u1a_ref, u1b_ref):
    s = jnp.concatenate([sa_ref[...], sb_ref[...]], axis=1)
    h1 = a_ref[...] * h_ref[...] + dis_ref[...] * s
    h1_ref[...] = h1
    u1a_ref[...] = dis_ref[...] * h1[:, :128]
    u1b_ref[...] = dis_ref[...] * h1[:, 128:]


def _combine_c(h, sa, sb, a2, dis2):
    return pl.pallas_call(
        _combine_c_body, grid=(_GRID,),
        in_specs=[_bs(256), _bs(128), _bs(128), _bs(1), _bs(1)],
        out_specs=[_bs(256), _bs(128), _bs(128)],
        out_shape=[jax.ShapeDtypeStruct((N, 256), jnp.float32)]
        + [jax.ShapeDtypeStruct((N, 128), jnp.float32)] * 2,
    )(h, sa, sb, a2, dis2)


def _final_c_body(h_ref, sa_ref, sb_ref, a_ref, dis_ref, z1_ref, t2_ref):
    z1_ref[...] = a_ref[...] * h_ref[:, :128] + dis_ref[...] * sa_ref[...]
    t2_ref[...] = a_ref[...] * h_ref[:, 128:] + dis_ref[...] * sb_ref[...]


def _final_c(h, sa, sb, a2, dis2):
    return pl.pallas_call(
        _final_c_body, grid=(_GRID,),
        in_specs=[_bs(256), _bs(128), _bs(128), _bs(1), _bs(1)],
        out_specs=[_bs(128), _bs(128)],
        out_shape=[jax.ShapeDtypeStruct((N, 128), jnp.float32)] * 2,
    )(h, sa, sb, a2, dis2)


def _combine_s_body(h_ref, s_ref, a_ref, dis_ref, h1_ref, u1_ref):
    h1 = a_ref[...] * h_ref[...] + dis_ref[...] * s_ref[...]
    h1_ref[...] = h1
    u1_ref[...] = dis_ref[...] * h1


def _combine_s(h, s, a2, dis2):
    return pl.pallas_call(
        _combine_s_body, grid=(_GRID,),
        in_specs=[_bs(128), _bs(128), _bs(1), _bs(1)],
        out_specs=[_bs(128), _bs(128)],
        out_shape=[jax.ShapeDtypeStruct((N, 128), jnp.float32)] * 2,
    )(h, s, a2, dis2)


def _final_s_body(h_ref, s_ref, a_ref, dis_ref, t2_ref, z2_ref):
    z2_ref[...] = (a_ref[...] * h_ref[...] + dis_ref[...] * s_ref[...]
                   + t2_ref[...])


def _final_s(h, s, a2, dis2, t2):
    return pl.pallas_call(
        _final_s_body, grid=(_GRID,),
        in_specs=[_bs(128), _bs(128), _bs(1), _bs(1), _bs(128)],
        out_specs=_bs(128),
        out_shape=jax.ShapeDtypeStruct((N, 128), jnp.float32),
    )(h, s, a2, dis2, t2)


def kernel(x, pri_edges, sup_edges, W1, b1, W2, b2):
    pe = pri_edges.astype(jnp.int32)
    se = sup_edges.astype(jnp.int32)
    prow, pcol = pe[0], pe[1]
    srow, scol = se[0], se[1]

    degp = _deg_kernel(pcol).reshape(1564, 128)
    degs = _deg_kernel(scol).reshape(1564, 128)
    disp, ap, diss, as_ = _norm_call(degp, degs)

    def col2d(v):
        return v.reshape(NP)[:N].reshape(N, 1)

    disp2, ap2, diss2, as2 = map(col2d, (disp, ap, diss, as_))

    wc = jnp.concatenate([W1, W2], axis=1)
    bc = jnp.concatenate([b1, b2]).reshape(1, 256)
    h0c, h0s, u0a, u0b, u0s = _mm_call(x, wc, bc, disp2, diss2)

    s1a, s1b, pcl, prl, pcn = _prop2_save(u0a, u0b, prow, pcol)
    h1c, u1a, u1b = _combine_c(h0c, s1a, s1b, ap2, disp2)
    s2a, s2b = _prop2_reuse(u1a, u1b, pcl, prl, pcn)
    z1, t2 = _final_c(h1c, s2a, s2b, ap2, disp2)

    s1s, scl, srl, scn = _prop1_save(u0s, srow, scol)
    h1s, u1s = _combine_s(h0s, s1s, as2, diss2)
    s2s = _prop1_reuse(u1s, scl, srl, scn)
    z2 = _final_s(h1s, s2s, as2, diss2, t2)
    return (z1, z2)


# speedup vs baseline: 1.2023x; 1.1746x over previous
import functools

import jax
import jax.numpy as jnp
from jax import lax
from jax.experimental import pallas as pl
from jax.experimental.pallas import tpu as pltpu, tpu_sc as plsc

N = 100000
E = 3200000
D_IN = 128
EPS = 0.5

NP = 100096
NTILES = 16
ECORE = E // (2 * NTILES)
ETILE = E // NTILES
EBLK = 1600

_mesh = lambda: plsc.VectorSubcoreMesh(core_axis_name="c", subcore_axis_name="s")


def _deg_body(col_hbm, out_hbm, idx_v, idxt_v, ones_v, zstage_v, acc_sh, sem):
    core = lax.axis_index("c")
    sub = lax.axis_index("s")
    zeros16 = jnp.zeros((16,), jnp.float32)
    ones16 = jnp.ones((16,), jnp.float32)
    for k in range(8):
        ones_v[pl.ds(16 * k, 16)] = ones16
    for k in range(64):
        zstage_v[pl.ds(16 * k, 16)] = zeros16
    zbase = sub * (NP // NTILES)
    for off in (0, 1024, 2048, 3072, 4096, 5120):
        pltpu.sync_copy(zstage_v, acc_sh.at[pl.ds(zbase + off, 1024)])
    pltpu.sync_copy(zstage_v.at[pl.ds(0, 112)], acc_sh.at[pl.ds(zbase + 6144, 112)])
    plsc.subcore_barrier()

    base_e = (core * NTILES + sub) * ECORE

    def body(b, _):
        pltpu.sync_copy(col_hbm.at[pl.ds(base_e + b * 128, 128)], idx_v)
        pltpu.sync_copy(ones_v, acc_sh.at[idx_v], add=True)
        return 0

    lax.fori_loop(0, ECORE // 128, body, 0)
    pltpu.sync_copy(col_hbm.at[pl.ds(base_e + 781 * 128, 32)], idxt_v)
    pltpu.sync_copy(ones_v.at[pl.ds(0, 32)], acc_sh.at[idxt_v], add=True)
    plsc.subcore_barrier()

    @pl.when(sub == 0)
    def _():
        pltpu.sync_copy(acc_sh, out_hbm.at[core])


_deg_kernel = functools.partial(
    pl.kernel,
    mesh=_mesh(),
    out_type=jax.ShapeDtypeStruct((2, NP), jnp.float32),
    scratch_types=[
        pltpu.VMEM((128,), jnp.int32),
        pltpu.VMEM((32,), jnp.int32),
        pltpu.VMEM((128,), jnp.float32),
        pltpu.VMEM((1024,), jnp.float32),
        pltpu.VMEM_SHARED((NP,), jnp.float32),
        pltpu.SemaphoreType.DMA,
    ],
)(_deg_body)


PC = 7168
PASSES = 7
PCAP = 15872
SROWS = PASSES * 2 * PC
TPT = PC // NTILES


def _make_propagate(HALVES, mode):
    def body(*refs):
        i = HALVES
        u_hbms = refs[:HALVES]
        if mode == "save":
            row_hbm, col_hbm = refs[i], refs[i + 1]
            i += 2
        else:
            cl_hbm, rl_hbm, cn_hbm = refs[i], refs[i + 1], refs[i + 2]
            i += 3
        s_hbms = refs[i:i + HALVES]
        i += HALVES
        if mode == "save":
            cl_hbm, rl_hbm, cn_hbm = refs[i], refs[i + 1], refs[i + 2]
            i += 3
        (cstage, rstage, cbuf, rbuf, staging, zbuf, idxg, idxs0, idxs1,
         cntbuf, acc, gsem0, gsem1, ssem0, ssem1, esem0, esem1) = refs[i:]
        core = lax.axis_index("c")
        sub = lax.axis_index("s")
        zeros16 = jnp.zeros((16,), jnp.float32)
        for r in range(8):
            for f in range(8):
                zbuf[r, pl.ds(16 * f, 16)] = zeros16
        zoff = sub * TPT

        def zero_slice(j, _):
            pltpu.sync_copy(zbuf, acc.at[pl.ds(zoff + 8 * j, 8)])
            return 0

        lax.fori_loop(0, TPT // 8, zero_slice, 0)
        if mode == "reuse":
            pltpu.sync_copy(cn_hbm.at[core].at[sub], cntbuf)
        plsc.subcore_barrier()

        def one_pass(p, _):
            base = (p * 2 + core) * PC

            if mode == "save":
                ebase = sub * ETILE

                esems = (esem0, esem1)

                def fire_e(b, slot):
                    off = ebase + b * EBLK
                    pltpu.async_copy(col_hbm.at[pl.ds(off, EBLK)],
                                     cstage.at[pl.ds(slot * EBLK, EBLK)],
                                     esems[slot])
                    pltpu.async_copy(row_hbm.at[pl.ds(off, EBLK)],
                                     rstage.at[pl.ds(slot * EBLK, EBLK)],
                                     esems[slot])

                def wait_e(b, slot):
                    off = ebase + b * EBLK
                    pltpu.make_async_copy(
                        col_hbm.at[pl.ds(off, EBLK)],
                        cstage.at[pl.ds(slot * EBLK, EBLK)],
                        esems[slot]).wait()
                    pltpu.make_async_copy(
                        row_hbm.at[pl.ds(off, EBLK)],
                        rstage.at[pl.ds(slot * EBLK, EBLK)],
                        esems[slot]).wait()

                fire_e(0, 0)

                def blk_body(b, cntv):
                    s = b % 2

                    @pl.when(b + 1 < ETILE // EBLK)
                    def _():
                        @pl.when(s == 0)
                        def _():
                            fire_e(b + 1, 1)

                        @pl.when(s == 1)
                        def _():
                            fire_e(b + 1, 0)

                    @pl.when(s == 0)
                    def _():
                        wait_e(b, 0)

                    @pl.when(s == 1)
                    def _():
                        wait_e(b, 1)
                    soff = s * EBLK

                    def f_body(i, cntv):
                        for u in range(5):
                            c16 = cstage[pl.ds(soff + i * 80 + u * 16, 16)]
                            r16 = rstage[pl.ds(soff + i * 80 + u * 16, 16)]
                            lo = c16 - base
                            m = (lo >= 0) & (lo < PC)
                            m32 = m.astype(jnp.int32)
                            csum = plsc.cumsum(m32)
                            pos = csum - m32 + cntv
                            plsc.store_scatter(cbuf, [pos], lo, mask=m)
                            plsc.store_scatter(rbuf, [pos], r16, mask=m)
                            cntv = cntv + plsc.all_reduce_population_count(m)
                        return jnp.minimum(cntv, PCAP - 224)

                    return lax.fori_loop(0, EBLK // 80, f_body, cntv)

                cntv = lax.fori_loop(0, ETILE // EBLK, blk_body,
                                     jnp.zeros((16,), jnp.int32))
                lane0 = lax.iota(jnp.int32, 16)
                cnt = jnp.sum(jnp.where(lane0 == 0, cntv, 0))

                lane = lax.iota(jnp.int32, 16)
                wid = core * NTILES + sub
                padr = wid * 3000 + lane * 64
                padc = PC + lane
                for k in range(8):
                    cbuf[pl.ds(cnt + 16 * k, 16)] = padc
                    rbuf[pl.ds(cnt + 16 * k, 16)] = padr
                plsc.store_scatter(cntbuf, [jnp.full((16,), p, jnp.int32)],
                                   jnp.full((16,), cnt, jnp.int32),
                                   mask=lane == p)
                pltpu.sync_copy(cbuf, cl_hbm.at[core].at[sub].at[p])
                pltpu.sync_copy(rbuf, rl_hbm.at[core].at[sub].at[p])
            else:
                pltpu.sync_copy(cl_hbm.at[core].at[sub].at[p], cbuf)
                pltpu.sync_copy(rl_hbm.at[core].at[sub].at[p], rbuf)
                cv = cntbuf[pl.ds(0, 16)]
                cnt = jnp.sum(jnp.where(lax.iota(jnp.int32, 16) == p, cv, 0))
            nb = (cnt + 127) // 128

            for h in range(HALVES):
                u_hbm = u_hbms[h]
                gsems = (gsem0, gsem1)
                ssems = (ssem0, ssem1)
                sidx = (idxs0, idxs1)

                def load_gidx(b, slot):
                    for k in range(8):
                        idxg[pl.ds(slot * 128 + 16 * k, 16)] = (
                            rbuf[pl.ds(b * 128 + 16 * k, 16)])

                def fire_g(slot):
                    pltpu.async_copy(
                        u_hbm.at[idxg.at[pl.ds(slot * 128, 128)]],
                        staging.at[pl.ds(slot * 128, 128)], gsems[slot])

                def wait_g(slot):
                    pltpu.make_async_copy(
                        u_hbm.at[idxg.at[pl.ds(slot * 128, 128)]],
                        staging.at[pl.ds(slot * 128, 128)], gsems[slot]).wait()

                def fire_s(slot):
                    pltpu.async_copy(staging.at[pl.ds(slot * 128, 128)],
                                     acc.at[sidx[slot]], ssems[slot], add=True)

                def wait_s(slot):
                    pltpu.make_async_copy(staging.at[pl.ds(slot * 128, 128)],
                                          acc.at[sidx[slot]],
                                          ssems[slot]).wait()

                @pl.when(nb > 0)
                def _():
                    load_gidx(0, 0)
                    fire_g(0)

                def batch(b, _):
                    s = b % 2

                    @pl.when(b >= 1)
                    def _():
                        @pl.when(s == 0)
                        def _():
                            wait_s(1)

                        @pl.when(s == 1)
                        def _():
                            wait_s(0)

                    @pl.when(b + 1 < nb)
                    def _():
                        @pl.when(s == 0)
                        def _():
                            load_gidx(b + 1, 1)
                            fire_g(1)

                        @pl.when(s == 1)
                        def _():
                            load_gidx(b + 1, 0)
                            fire_g(0)

                    @pl.when(s == 0)
                    def _():
                        wait_g(0)
                        for k in range(8):
                            idxs0[pl.ds(16 * k, 16)] = (
                                cbuf[pl.ds(b * 128 + 16 * k, 16)])
                        fire_s(0)

                    @pl.when(s == 1)
                    def _():
                        wait_g(1)
                        for k in range(8):
                            idxs1[pl.ds(16 * k, 16)] = (
                                cbuf[pl.ds(b * 128 + 16 * k, 16)])
                        fire_s(1)
                    return 0

                lax.fori_loop(0, nb, batch, 0)

                @pl.when(nb > 0)
                def _():
                    @pl.when((nb - 1) % 2 == 0)
                    def _():
                        wait_s(0)

                    @pl.when((nb - 1) % 2 == 1)
                    def _():
                        wait_s(1)

                plsc.subcore_barrier()
                pltpu.sync_copy(acc.at[pl.ds(zoff, TPT)],
                                s_hbms[h].at[pl.ds(base + zoff, TPT)])
                lax.fori_loop(0, TPT // 8, zero_slice, 0)
                plsc.subcore_barrier()
            return 0

        lax.fori_loop(0, PASSES, one_pass, 0)
        if mode == "save":
            pltpu.sync_copy(cntbuf, cn_hbm.at[core].at[sub])

    s_t = [jax.ShapeDtypeStruct((SROWS, 128), jnp.float32)] * HALVES
    lists_t = [
        jax.ShapeDtypeStruct((2, NTILES, 8, PCAP), jnp.int32),
        jax.ShapeDtypeStruct((2, NTILES, 8, PCAP), jnp.int32),
        jax.ShapeDtypeStruct((2, NTILES, 16), jnp.int32),
    ]
    out_t = s_t + lists_t if mode == "save" else (
        s_t if HALVES > 1 else s_t[0])
    return functools.partial(
        pl.kernel,
        mesh=_mesh(),
        out_type=out_t,
        scratch_types=[
            pltpu.VMEM((2 * EBLK,), jnp.int32),
            pltpu.VMEM((2 * EBLK,), jnp.int32),
            pltpu.VMEM((PCAP,), jnp.int32),
            pltpu.VMEM((PCAP,), jnp.int32),
            pltpu.VMEM((256, 128), jnp.float32),
            pltpu.VMEM((8, 128), jnp.float32),
            pltpu.VMEM((256,), jnp.int32),
            pltpu.VMEM((128,), jnp.int32),
            pltpu.VMEM((128,), jnp.int32),
            pltpu.VMEM((16,), jnp.int32),
            pltpu.VMEM_SHARED((PC + 16, 128), jnp.float32),
            pltpu.SemaphoreType.DMA,
            pltpu.SemaphoreType.DMA,
            pltpu.SemaphoreType.DMA,
            pltpu.SemaphoreType.DMA,
            pltpu.SemaphoreType.DMA,
            pltpu.SemaphoreType.DMA,
        ],
        compiler_params=pltpu.CompilerParams(needs_layout_passes=False),
    )(body)


_prop2_save = _make_propagate(2, "save")
_prop2_reuse = _make_propagate(2, "reuse")
_prop1_save = _make_propagate(1, "save")
_prop1_reuse = _make_propagate(1, "reuse")


_RB = 200
_GRID = N // _RB
_bs = lambda w: pl.BlockSpec((_RB, w), lambda i: (i, 0))


def _mm_body(x_ref, w_ref, b_ref, disp_ref, diss_ref,
             h0c_ref, h0s_ref, u0a_ref, u0b_ref, u0s_ref):
    h = jnp.dot(x_ref[...], w_ref[...], preferred_element_type=jnp.float32)
    h = h + b_ref[...]
    hb = h[:, 128:]
    h0c_ref[...] = h
    h0s_ref[...] = hb
    u0a_ref[...] = disp_ref[...] * h[:, :128]
    u0b_ref[...] = disp_ref[...] * hb
    u0s_ref[...] = diss_ref[...] * hb


def _mm_call(x, wc, bc, disp, diss):
    return pl.pallas_call(
        _mm_body,
        grid=(_GRID,),
        in_specs=[
            _bs(128),
            pl.BlockSpec((128, 256), lambda i: (0, 0)),
            pl.BlockSpec((1, 256), lambda i: (0, 0)),
            _bs(1),
            _bs(1),
        ],
        out_specs=[_bs(256), _bs(128), _bs(128), _bs(128), _bs(128)],
        out_shape=[
            jax.ShapeDtypeStruct((N, 256), jnp.float32),
            jax.ShapeDtypeStruct((N, 128), jnp.float32),
            jax.ShapeDtypeStruct((N, 128), jnp.float32),
            jax.ShapeDtypeStruct((N, 128), jnp.float32),
            jax.ShapeDtypeStruct((N, 128), jnp.float32),
        ],
    )(x, wc, bc, disp, diss)


def _norm_body(pp_ref, sp_ref, disp_ref, ap_ref, diss_ref, as_ref):
    for pref, dref, aref in ((pp_ref, disp_ref, ap_ref),
                             (sp_ref, diss_ref, as_ref)):
        deg = pref[:782, :] + pref[782:, :] + EPS
        dis = lax.rsqrt(deg)
        dref[...] = dis
        aref[...] = (1.0 - EPS) + EPS * dis * dis


def _norm_call(pparts, sparts):
    return pl.pallas_call(
        _norm_body,
        grid=(1,),
        in_specs=[pl.BlockSpec((1564, 128), lambda i: (0, 0))] * 2,
        out_specs=[pl.BlockSpec((782, 128), lambda i: (0, 0))] * 4,
        out_shape=[jax.ShapeDtypeStruct((782, 128), jnp.float32)] * 4,
    )(pparts, sparts)


def _combine_c_body(h_ref, sa_ref, sb_ref, a_ref, dis_ref,
                    h1_ref, u1a_ref, u1b_ref):
    s = jnp.concatenate([sa_ref[...], sb_ref[...]], axis=1)
    h1 = a_ref[...] * h_ref[...] + dis_ref[...] * s
    h1_ref[...] = h1
    u1a_ref[...] = dis_ref[...] * h1[:, :128]
    u1b_ref[...] = dis_ref[...] * h1[:, 128:]


def _combine_c(h, sa, sb, a2, dis2):
    return pl.pallas_call(
        _combine_c_body, grid=(_GRID,),
        in_specs=[_bs(256), _bs(128), _bs(128), _bs(1), _bs(1)],
        out_specs=[_bs(256), _bs(128), _bs(128)],
        out_shape=[jax.ShapeDtypeStruct((N, 256), jnp.float32)]
        + [jax.ShapeDtypeStruct((N, 128), jnp.float32)] * 2,
    )(h, sa, sb, a2, dis2)


def _final_c_body(h_ref, sa_ref, sb_ref, a_ref, dis_ref, z1_ref, t2_ref):
    z1_ref[...] = a_ref[...] * h_ref[:, :128] + dis_ref[...] * sa_ref[...]
    t2_ref[...] = a_ref[...] * h_ref[:, 128:] + dis_ref[...] * sb_ref[...]


def _final_c(h, sa, sb, a2, dis2):
    return pl.pallas_call(
        _final_c_body, grid=(_GRID,),
        in_specs=[_bs(256), _bs(128), _bs(128), _bs(1), _bs(1)],
        out_specs=[_bs(128), _bs(128)],
        out_shape=[jax.ShapeDtypeStruct((N, 128), jnp.float32)] * 2,
    )(h, sa, sb, a2, dis2)


def _combine_s_body(h_ref, s_ref, a_ref, dis_ref, h1_ref, u1_ref):
    h1 = a_ref[...] * h_ref[...] + dis_ref[...] * s_ref[...]
    h1_ref[...] = h1
    u1_ref[...] = dis_ref[...] * h1


def _combine_s(h, s, a2, dis2):
    return pl.pallas_call(
        _combine_s_body, grid=(_GRID,),
        in_specs=[_bs(128), _bs(128), _bs(1), _bs(1)],
        out_specs=[_bs(128), _bs(128)],
        out_shape=[jax.ShapeDtypeStruct((N, 128), jnp.float32)] * 2,
    )(h, s, a2, dis2)


def _final_s_body(h_ref, s_ref, a_ref, dis_ref, t2_ref, z2_ref):
    z2_ref[...] = (a_ref[...] * h_ref[...] + dis_ref[...] * s_ref[...]
                   + t2_ref[...])


def _final_s(h, s, a2, dis2, t2):
    return pl.pallas_call(
        _final_s_body, grid=(_GRID,),
        in_specs=[_bs(128), _bs(128), _bs(1), _bs(1), _bs(128)],
        out_specs=_bs(128),
        out_shape=jax.ShapeDtypeStruct((N, 128), jnp.float32),
    )(h, s, a2, dis2, t2)


def kernel(x, pri_edges, sup_edges, W1, b1, W2, b2):
    pe = pri_edges.astype(jnp.int32)
    se = sup_edges.astype(jnp.int32)
    prow, pcol = pe[0], pe[1]
    srow, scol = se[0], se[1]

    degp = _deg_kernel(pcol).reshape(1564, 128)
    degs = _deg_kernel(scol).reshape(1564, 128)
    disp, ap, diss, as_ = _norm_call(degp, degs)

    def col2d(v):
        return v.reshape(NP)[:N].reshape(N, 1)

    disp2, ap2, diss2, as2 = map(col2d, (disp, ap, diss, as_))

    wc = jnp.concatenate([W1, W2], axis=1)
    bc = jnp.concatenate([b1, b2]).reshape(1, 256)
    h0c, h0s, u0a, u0b, u0s = _mm_call(x, wc, bc, disp2, diss2)

    s1a, s1b, pcl, prl, pcn = _prop2_save(u0a, u0b, prow, pcol)
    h1c, u1a, u1b = _combine_c(h0c, s1a, s1b, ap2, disp2)
    s2a, s2b = _prop2_reuse(u1a, u1b, pcl, prl, pcn)
    z1, t2 = _final_c(h1c, s2a, s2b, ap2, disp2)

    s1s, scl, srl, scn = _prop1_save(u0s, srow, scol)
    h1s, u1s = _combine_s(h0s, s1s, as2, diss2)
    s2s = _prop1_reuse(u1s, scl, srl, scn)
    z2 = _final_s(h1s, s2s, as2, diss2, t2)
    return (z1, z2)


# speedup vs baseline: 1.2583x; 1.0466x over previous
import functools

import jax
import jax.numpy as jnp
from jax import lax
from jax.experimental import pallas as pl
from jax.experimental.pallas import tpu as pltpu, tpu_sc as plsc

N = 100000
E = 3200000
D_IN = 128
EPS = 0.5

NP = 100096
NTILES = 16
ECORE = E // (2 * NTILES)
ETILE = E // NTILES
EBLK = 1600

_mesh = lambda: plsc.VectorSubcoreMesh(core_axis_name="c", subcore_axis_name="s")


def _deg_body(col_hbm, out_hbm, idx_a, idx_b, idxt_v, ones_v, zstage_v,
              acc_sh, dsem0, dsem1):
    core = lax.axis_index("c")
    sub = lax.axis_index("s")
    zeros16 = jnp.zeros((16,), jnp.float32)
    ones16 = jnp.ones((16,), jnp.float32)
    for k in range(8):
        ones_v[pl.ds(16 * k, 16)] = ones16
    for k in range(64):
        zstage_v[pl.ds(16 * k, 16)] = zeros16
    zbase = sub * (NP // NTILES)
    for off in (0, 1024, 2048, 3072, 4096, 5120):
        pltpu.sync_copy(zstage_v, acc_sh.at[pl.ds(zbase + off, 1024)])
    pltpu.sync_copy(zstage_v.at[pl.ds(0, 112)], acc_sh.at[pl.ds(zbase + 6144, 112)])
    plsc.subcore_barrier()

    base_e = (core * NTILES + sub) * ECORE
    slots = ((idx_a, dsem0), (idx_b, dsem1))

    def fire_i(b, slot):
        ref, sem = slots[slot]
        pltpu.async_copy(col_hbm.at[pl.ds(base_e + b * 128, 128)], ref, sem)

    def wait_i(b, slot):
        ref, sem = slots[slot]
        pltpu.make_async_copy(col_hbm.at[pl.ds(base_e + b * 128, 128)],
                              ref, sem).wait()

    fire_i(0, 0)

    def body(b, _):
        s = b % 2

        @pl.when(b + 1 < ECORE // 128)
        def _():
            @pl.when(s == 0)
            def _():
                fire_i(b + 1, 1)

            @pl.when(s == 1)
            def _():
                fire_i(b + 1, 0)

        @pl.when(s == 0)
        def _():
            wait_i(b, 0)
            pltpu.sync_copy(ones_v, acc_sh.at[idx_a], add=True)

        @pl.when(s == 1)
        def _():
            wait_i(b, 1)
            pltpu.sync_copy(ones_v, acc_sh.at[idx_b], add=True)
        return 0

    lax.fori_loop(0, ECORE // 128, body, 0)
    pltpu.sync_copy(col_hbm.at[pl.ds(base_e + 781 * 128, 32)], idxt_v)
    pltpu.sync_copy(ones_v.at[pl.ds(0, 32)], acc_sh.at[idxt_v], add=True)
    plsc.subcore_barrier()

    @pl.when(sub == 0)
    def _():
        pltpu.sync_copy(acc_sh, out_hbm.at[core])


_deg_kernel = functools.partial(
    pl.kernel,
    mesh=_mesh(),
    out_type=jax.ShapeDtypeStruct((2, NP), jnp.float32),
    scratch_types=[
        pltpu.VMEM((128,), jnp.int32),
        pltpu.VMEM((128,), jnp.int32),
        pltpu.VMEM((32,), jnp.int32),
        pltpu.VMEM((128,), jnp.float32),
        pltpu.VMEM((1024,), jnp.float32),
        pltpu.VMEM_SHARED((NP,), jnp.float32),
        pltpu.SemaphoreType.DMA,
        pltpu.SemaphoreType.DMA,
    ],
)(_deg_body)


PC = 7168
PASSES = 7
PCAP = 15872
SROWS = PASSES * 2 * PC
TPT = PC // NTILES


def _make_propagate(HALVES, mode):
    def body(*refs):
        i = HALVES
        u_hbms = refs[:HALVES]
        if mode == "save":
            row_hbm, col_hbm = refs[i], refs[i + 1]
            i += 2
        else:
            cl_hbm, rl_hbm, cn_hbm = refs[i], refs[i + 1], refs[i + 2]
            i += 3
        s_hbms = refs[i:i + HALVES]
        i += HALVES
        if mode == "save":
            cl_hbm, rl_hbm, cn_hbm = refs[i], refs[i + 1], refs[i + 2]
            i += 3
        (cstage, rstage, cbuf, rbuf, staging, zbuf, idxg, idxs0, idxs1,
         cntbuf, acc, gsem0, gsem1, ssem0, ssem1, esem0, esem1) = refs[i:]
        core = lax.axis_index("c")
        sub = lax.axis_index("s")
        zeros16 = jnp.zeros((16,), jnp.float32)
        for r in range(8):
            for f in range(8):
                zbuf[r, pl.ds(16 * f, 16)] = zeros16
        zoff = sub * TPT

        def zero_slice(j, _):
            pltpu.sync_copy(zbuf, acc.at[pl.ds(zoff + 8 * j, 8)])
            return 0

        lax.fori_loop(0, TPT // 8, zero_slice, 0)
        if mode == "reuse":
            pltpu.sync_copy(cn_hbm.at[core].at[sub], cntbuf)
        plsc.subcore_barrier()

        def one_pass(p, _):
            base = (p * 2 + core) * PC

            if mode == "save":
                ebase = sub * ETILE

                esems = (esem0, esem1)

                def fire_e(b, slot):
                    off = ebase + b * EBLK
                    pltpu.async_copy(col_hbm.at[pl.ds(off, EBLK)],
                                     cstage.at[pl.ds(slot * EBLK, EBLK)],
                                     esems[slot])
                    pltpu.async_copy(row_hbm.at[pl.ds(off, EBLK)],
                                     rstage.at[pl.ds(slot * EBLK, EBLK)],
                                     esems[slot])

                def wait_e(b, slot):
                    off = ebase + b * EBLK
                    pltpu.make_async_copy(
                        col_hbm.at[pl.ds(off, EBLK)],
                        cstage.at[pl.ds(slot * EBLK, EBLK)],
                        esems[slot]).wait()
                    pltpu.make_async_copy(
                        row_hbm.at[pl.ds(off, EBLK)],
                        rstage.at[pl.ds(slot * EBLK, EBLK)],
                        esems[slot]).wait()

                fire_e(0, 0)

                def blk_body(b, cntv):
                    s = b % 2

                    @pl.when(b + 1 < ETILE // EBLK)
                    def _():
                        @pl.when(s == 0)
                        def _():
                            fire_e(b + 1, 1)

                        @pl.when(s == 1)
                        def _():
                            fire_e(b + 1, 0)

                    @pl.when(s == 0)
                    def _():
                        wait_e(b, 0)

                    @pl.when(s == 1)
                    def _():
                        wait_e(b, 1)
                    soff = s * EBLK

                    def f_body(i, cntv):
                        for u in range(5):
                            c16 = cstage[pl.ds(soff + i * 80 + u * 16, 16)]
                            r16 = rstage[pl.ds(soff + i * 80 + u * 16, 16)]
                            lo = c16 - base
                            m = (lo >= 0) & (lo < PC)
                            m32 = m.astype(jnp.int32)
                            csum = plsc.cumsum(m32)
                            pos = csum - m32 + cntv
                            plsc.store_scatter(cbuf, [pos], lo, mask=m)
                            plsc.store_scatter(rbuf, [pos], r16, mask=m)
                            cntv = cntv + plsc.all_reduce_population_count(m)
                        return jnp.minimum(cntv, PCAP - 224)

                    return lax.fori_loop(0, EBLK // 80, f_body, cntv)

                cntv = lax.fori_loop(0, ETILE // EBLK, blk_body,
                                     jnp.zeros((16,), jnp.int32))
                lane0 = lax.iota(jnp.int32, 16)
                cnt = jnp.sum(jnp.where(lane0 == 0, cntv, 0))

                lane = lax.iota(jnp.int32, 16)
                wid = core * NTILES + sub
                padr = wid * 3000 + lane * 64
                padc = PC + lane
                for k in range(8):
                    cbuf[pl.ds(cnt + 16 * k, 16)] = padc
                    rbuf[pl.ds(cnt + 16 * k, 16)] = padr
                plsc.store_scatter(cntbuf, [jnp.full((16,), p, jnp.int32)],
                                   jnp.full((16,), cnt, jnp.int32),
                                   mask=lane == p)
                pltpu.sync_copy(cbuf, cl_hbm.at[core].at[sub].at[p])
                pltpu.sync_copy(rbuf, rl_hbm.at[core].at[sub].at[p])
            else:
                pltpu.sync_copy(cl_hbm.at[core].at[sub].at[p], cbuf)
                pltpu.sync_copy(rl_hbm.at[core].at[sub].at[p], rbuf)
                cv = cntbuf[pl.ds(0, 16)]
                cnt = jnp.sum(jnp.where(lax.iota(jnp.int32, 16) == p, cv, 0))
            nb = (cnt + 127) // 128

            for h in range(HALVES):
                u_hbm = u_hbms[h]
                gsems = (gsem0, gsem1)
                ssems = (ssem0, ssem1)
                sidx = (idxs0, idxs1)

                def load_gidx(b, slot):
                    for k in range(8):
                        idxg[pl.ds(slot * 128 + 16 * k, 16)] = (
                            rbuf[pl.ds(b * 128 + 16 * k, 16)])

                def fire_g(slot):
                    pltpu.async_copy(
                        u_hbm.at[idxg.at[pl.ds(slot * 128, 128)]],
                        staging.at[pl.ds(slot * 128, 128)], gsems[slot])

                def wait_g(slot):
                    pltpu.make_async_copy(
                        u_hbm.at[idxg.at[pl.ds(slot * 128, 128)]],
                        staging.at[pl.ds(slot * 128, 128)], gsems[slot]).wait()

                def fire_s(slot):
                    pltpu.async_copy(staging.at[pl.ds(slot * 128, 128)],
                                     acc.at[sidx[slot]], ssems[slot], add=True)

                def wait_s(slot):
                    pltpu.make_async_copy(staging.at[pl.ds(slot * 128, 128)],
                                          acc.at[sidx[slot]],
                                          ssems[slot]).wait()

                @pl.when(nb > 0)
                def _():
                    load_gidx(0, 0)
                    fire_g(0)

                def batch(b, _):
                    s = b % 2

                    @pl.when(b >= 1)
                    def _():
                        @pl.when(s == 0)
                        def _():
                            wait_s(1)

                        @pl.when(s == 1)
                        def _():
                            wait_s(0)

                    @pl.when(b + 1 < nb)
                    def _():
                        @pl.when(s == 0)
                        def _():
                            load_gidx(b + 1, 1)
                            fire_g(1)

                        @pl.when(s == 1)
                        def _():
                            load_gidx(b + 1, 0)
                            fire_g(0)

                    @pl.when(s == 0)
                    def _():
                        wait_g(0)
                        for k in range(8):
                            idxs0[pl.ds(16 * k, 16)] = (
                                cbuf[pl.ds(b * 128 + 16 * k, 16)])
                        fire_s(0)

                    @pl.when(s == 1)
                    def _():
                        wait_g(1)
                        for k in range(8):
                            idxs1[pl.ds(16 * k, 16)] = (
                                cbuf[pl.ds(b * 128 + 16 * k, 16)])
                        fire_s(1)
                    return 0

                lax.fori_loop(0, nb, batch, 0)

                @pl.when(nb > 0)
                def _():
                    @pl.when((nb - 1) % 2 == 0)
                    def _():
                        wait_s(0)

                    @pl.when((nb - 1) % 2 == 1)
                    def _():
                        wait_s(1)

                plsc.subcore_barrier()
                pltpu.sync_copy(acc.at[pl.ds(zoff, TPT)],
                                s_hbms[h].at[pl.ds(base + zoff, TPT)])
                lax.fori_loop(0, TPT // 8, zero_slice, 0)
                plsc.subcore_barrier()
            return 0

        lax.fori_loop(0, PASSES, one_pass, 0)
        if mode == "save":
            pltpu.sync_copy(cntbuf, cn_hbm.at[core].at[sub])

    s_t = [jax.ShapeDtypeStruct((SROWS, 128), jnp.float32)] * HALVES
    lists_t = [
        jax.ShapeDtypeStruct((2, NTILES, 8, PCAP), jnp.int32),
        jax.ShapeDtypeStruct((2, NTILES, 8, PCAP), jnp.int32),
        jax.ShapeDtypeStruct((2, NTILES, 16), jnp.int32),
    ]
    out_t = s_t + lists_t if mode == "save" else (
        s_t if HALVES > 1 else s_t[0])
    return functools.partial(
        pl.kernel,
        mesh=_mesh(),
        out_type=out_t,
        scratch_types=[
            pltpu.VMEM((2 * EBLK,), jnp.int32),
            pltpu.VMEM((2 * EBLK,), jnp.int32),
            pltpu.VMEM((PCAP,), jnp.int32),
            pltpu.VMEM((PCAP,), jnp.int32),
            pltpu.VMEM((256, 128), jnp.float32),
            pltpu.VMEM((8, 128), jnp.float32),
            pltpu.VMEM((256,), jnp.int32),
            pltpu.VMEM((128,), jnp.int32),
            pltpu.VMEM((128,), jnp.int32),
            pltpu.VMEM((16,), jnp.int32),
            pltpu.VMEM_SHARED((PC + 16, 128), jnp.float32),
            pltpu.SemaphoreType.DMA,
            pltpu.SemaphoreType.DMA,
            pltpu.SemaphoreType.DMA,
            pltpu.SemaphoreType.DMA,
            pltpu.SemaphoreType.DMA,
            pltpu.SemaphoreType.DMA,
        ],
        compiler_params=pltpu.CompilerParams(needs_layout_passes=False),
    )(body)


_prop2_save = _make_propagate(2, "save")
_prop2_reuse = _make_propagate(2, "reuse")
_prop1_save = _make_propagate(1, "save")
_prop1_reuse = _make_propagate(1, "reuse")


_RB = 200
_GRID = N // _RB
_bs = lambda w: pl.BlockSpec((_RB, w), lambda i: (i, 0))


def _mm_body(x_ref, w_ref, b_ref, disp_ref, diss_ref,
             h0c_ref, h0s_ref, u0a_ref, u0b_ref, u0s_ref):
    h = jnp.dot(x_ref[...], w_ref[...], preferred_element_type=jnp.float32)
    h = h + b_ref[...]
    hb = h[:, 128:]
    h0c_ref[...] = h
    h0s_ref[...] = hb
    u0a_ref[...] = disp_ref[...] * h[:, :128]
    u0b_ref[...] = disp_ref[...] * hb
    u0s_ref[...] = diss_ref[...] * hb


def _mm_call(x, wc, bc, disp, diss):
    return pl.pallas_call(
        _mm_body,
        grid=(_GRID,),
        in_specs=[
            _bs(128),
            pl.BlockSpec((128, 256), lambda i: (0, 0)),
            pl.BlockSpec((1, 256), lambda i: (0, 0)),
            _bs(1),
            _bs(1),
        ],
        out_specs=[_bs(256), _bs(128), _bs(128), _bs(128), _bs(128)],
        out_shape=[
            jax.ShapeDtypeStruct((N, 256), jnp.float32),
            jax.ShapeDtypeStruct((N, 128), jnp.float32),
            jax.ShapeDtypeStruct((N, 128), jnp.float32),
            jax.ShapeDtypeStruct((N, 128), jnp.float32),
            jax.ShapeDtypeStruct((N, 128), jnp.float32),
        ],
    )(x, wc, bc, disp, diss)


def _norm_body(pp_ref, sp_ref, disp_ref, ap_ref, diss_ref, as_ref):
    for pref, dref, aref in ((pp_ref, disp_ref, ap_ref),
                             (sp_ref, diss_ref, as_ref)):
        deg = pref[:782, :] + pref[782:, :] + EPS
        dis = lax.rsqrt(deg)
        dref[...] = dis
        aref[...] = (1.0 - EPS) + EPS * dis * dis


def _norm_call(pparts, sparts):
    return pl.pallas_call(
        _norm_body,
        grid=(1,),
        in_specs=[pl.BlockSpec((1564, 128), lambda i: (0, 0))] * 2,
        out_specs=[pl.BlockSpec((782, 128), lambda i: (0, 0))] * 4,
        out_shape=[jax.ShapeDtypeStruct((782, 128), jnp.float32)] * 4,
    )(pparts, sparts)


def _combine_c_body(h_ref, sa_ref, sb_ref, a_ref, dis_ref,
                    h1_ref, u1a_ref, u1b_ref):
    s = jnp.concatenate([sa_ref[...], sb_ref[...]], axis=1)
    h1 = a_ref[...] * h_ref[...] + dis_ref[...] * s
    h1_ref[...] = h1
    u1a_ref[...] = dis_ref[...] * h1[:, :128]
    u1b_ref[...] = dis_ref[...] * h1[:, 128:]


def _combine_c(h, sa, sb, a2, dis2):
    return pl.pallas_call(
        _combine_c_body, grid=(_GRID,),
        in_specs=[_bs(256), _bs(128), _bs(128), _bs(1), _bs(1)],
        out_specs=[_bs(256), _bs(128), _bs(128)],
        out_shape=[jax.ShapeDtypeStruct((N, 256), jnp.float32)]
        + [jax.ShapeDtypeStruct((N, 128), jnp.float32)] * 2,
    )(h, sa, sb, a2, dis2)


def _final_c_body(h_ref, sa_ref, sb_ref, a_ref, dis_ref, z1_ref, t2_ref):
    z1_ref[...] = a_ref[...] * h_ref[:, :128] + dis_ref[...] * sa_ref[...]
    t2_ref[...] = a_ref[...] * h_ref[:, 128:] + dis_ref[...] * sb_ref[...]


def _final_c(h, sa, sb, a2, dis2):
    return pl.pallas_call(
        _final_c_body, grid=(_GRID,),
        in_specs=[_bs(256), _bs(128), _bs(128), _bs(1), _bs(1)],
        out_specs=[_bs(128), _bs(128)],
        out_shape=[jax.ShapeDtypeStruct((N, 128), jnp.float32)] * 2,
    )(h, sa, sb, a2, dis2)


def _combine_s_body(h_ref, s_ref, a_ref, dis_ref, h1_ref, u1_ref):
    h1 = a_ref[...] * h_ref[...] + dis_ref[...] * s_ref[...]
    h1_ref[...] = h1
    u1_ref[...] = dis_ref[...] * h1


def _combine_s(h, s, a2, dis2):
    return pl.pallas_call(
        _combine_s_body, grid=(_GRID,),
        in_specs=[_bs(128), _bs(128), _bs(1), _bs(1)],
        out_specs=[_bs(128), _bs(128)],
        out_shape=[jax.ShapeDtypeStruct((N, 128), jnp.float32)] * 2,
    )(h, s, a2, dis2)


def _final_s_body(h_ref, s_ref, a_ref, dis_ref, t2_ref, z2_ref):
    z2_ref[...] = (a_ref[...] * h_ref[...] + dis_ref[...] * s_ref[...]
                   + t2_ref[...])


def _final_s(h, s, a2, dis2, t2):
    return pl.pallas_call(
        _final_s_body, grid=(_GRID,),
        in_specs=[_bs(128), _bs(128), _bs(1), _bs(1), _bs(128)],
        out_specs=_bs(128),
        out_shape=jax.ShapeDtypeStruct((N, 128), jnp.float32),
    )(h, s, a2, dis2, t2)


def kernel(x, pri_edges, sup_edges, W1, b1, W2, b2):
    pe = pri_edges.astype(jnp.int32)
    se = sup_edges.astype(jnp.int32)
    prow, pcol = pe[0], pe[1]
    srow, scol = se[0], se[1]

    degp = _deg_kernel(pcol).reshape(1564, 128)
    degs = _deg_kernel(scol).reshape(1564, 128)
    disp, ap, diss, as_ = _norm_call(degp, degs)

    def col2d(v):
        return v.reshape(NP)[:N].reshape(N, 1)

    disp2, ap2, diss2, as2 = map(col2d, (disp, ap, diss, as_))

    wc = jnp.concatenate([W1, W2], axis=1)
    bc = jnp.concatenate([b1, b2]).reshape(1, 256)
    h0c, h0s, u0a, u0b, u0s = _mm_call(x, wc, bc, disp2, diss2)

    s1a, s1b, pcl, prl, pcn = _prop2_save(u0a, u0b, prow, pcol)
    h1c, u1a, u1b = _combine_c(h0c, s1a, s1b, ap2, disp2)
    s2a, s2b = _prop2_reuse(u1a, u1b, pcl, prl, pcn)
    z1, t2 = _final_c(h1c, s2a, s2b, ap2, disp2)

    s1s, scl, srl, scn = _prop1_save(u0s, srow, scol)
    h1s, u1s = _combine_s(h0s, s1s, as2, diss2)
    s2s = _prop1_reuse(u1s, scl, srl, scn)
    z2 = _final_s(h1s, s2s, as2, diss2, t2)
    return (z1, z2)
